# Initial kernel scaffold; baseline (speedup 1.0000x reference)
import jax
import jax.numpy as jnp
from jax.experimental import pallas as pl


def kernel(comp_feature, prot_feature, batch_comp, batch_prot, Wc, bc, Ws, bs, Wp, bp, Wr1, br1, Wr2, br2, Wa1, ba1, Wa2, ba2, Wo1, bo1, Wo2, bo2):
    raise NotImplementedError("write your pallas kernel here")


# speedup vs baseline: 2.2768x; 2.2768x over previous
import functools

import jax
import jax.numpy as jnp
from jax import lax
from jax.experimental import pallas as pl
from jax.experimental.pallas import tpu as pltpu

B_SEG = 1024
CH = 512
NEG_INF = float("-inf")


def _leaky(x):
    return jnp.where(x >= 0, x, 0.1 * x)


def _dot(a, b):
    return jnp.dot(a, b, preferred_element_type=jnp.float32)


def _prot_kernel(lo_ref, hi_ref, x_ref, bid_ref, wpT_ref, bp_ref, out_ref):
    i = pl.program_id(0)

    @pl.when(i == 0)
    def _():
        out_ref[...] = jnp.full_like(out_ref, NEG_INF)

    x = x_ref[...].astype(jnp.bfloat16)
    emb = _leaky(_dot(x, wpT_ref[...]) + bp_ref[...])
    bid = bid_ref[...]
    lo = lo_ref[i]
    hi = hi_ref[i]

    def body(seg, _):
        mask = bid == seg
        vals = jnp.where(mask, emb, NEG_INF)
        m = jnp.max(vals, axis=0, keepdims=True)
        out_ref[pl.ds(seg, 1), :] = jnp.maximum(out_ref[pl.ds(seg, 1), :], m)
        return 0

    lax.fori_loop(lo, hi + 1, body, 0)


def _sup_kernel(lo_ref, hi_ref, x_ref, bid_ref, wsT_ref, bs_ref, out_ref):
    i = pl.program_id(0)

    @pl.when(i == 0)
    def _():
        out_ref[...] = jnp.zeros_like(out_ref)

    x = x_ref[...].astype(jnp.bfloat16)
    s = _leaky(_dot(x, wsT_ref[...]) + bs_ref[...])
    bid = bid_ref[...]
    lo = lo_ref[i]
    hi = hi_ref[i]

    def body(seg, _):
        mask = bid == seg
        vals = jnp.where(mask, s, 0.0)
        acc = jnp.sum(vals, axis=0, keepdims=True)
        out_ref[pl.ds(seg, 1), :] = out_ref[pl.ds(seg, 1), :] + acc
        return 0

    lax.fori_loop(lo, hi + 1, body, 0)


def _main_kernel(lo_ref, hi_ref,
                 x_ref, bid_ref,
                 wcT_ref, bc_ref,
                 wr1cT_ref, wr1sT_ref, wr1pT_ref, br1_ref, wr2T_ref, br2_ref,
                 wa1cT_ref, wa1sT_ref, wa1pT_ref, ba1_ref, wa2T_ref, ba2_ref,
                 wo1T_ref, bo1_ref, wo2T_ref, bo2_ref,
                 sup_ref, prot_ref,
                 pa_out, vec_out, aff_out, m_out, r_out,
                 baser_s, basea_s, v_s, m_s, s_s):
    i = pl.program_id(0)
    nb = pl.num_programs(0)

    @pl.when(i == 0)
    def _():
        supb = sup_ref[...].astype(jnp.bfloat16)
        protb = prot_ref[...].astype(jnp.bfloat16)
        baser_s[...] = (_dot(supb, wr1sT_ref[...]) + _dot(protb, wr1pT_ref[...])
                        + br1_ref[...])
        basea_s[...] = (_dot(supb, wa1sT_ref[...]) + _dot(protb, wa1pT_ref[...])
                        + ba1_ref[...])
        v_s[...] = jnp.zeros_like(v_s)
        m_s[...] = jnp.full_like(m_s, NEG_INF)
        s_s[...] = jnp.zeros_like(s_s)

    x = x_ref[...].astype(jnp.bfloat16)
    emb = _leaky(_dot(x, wcT_ref[...]) + bc_ref[...])
    embb = emb.astype(jnp.bfloat16)
    bid = bid_ref[...]
    lo = lo_ref[i]
    hi = hi_ref[i]

    zero = jnp.zeros((CH, 128), jnp.float32)

    def bbody(seg, carry):
        br_, ba_ = carry
        mask = bid == seg
        br_ = jnp.where(mask, baser_s[pl.ds(seg, 1), :], br_)
        ba_ = jnp.where(mask, basea_s[pl.ds(seg, 1), :], ba_)
        return br_, ba_

    base_r, base_a = lax.fori_loop(lo, hi + 1, bbody, (zero, zero))

    h_r = _leaky(_dot(embb, wr1cT_ref[...]) + base_r)
    raw = _dot(h_r.astype(jnp.bfloat16), wr2T_ref[...]) + br2_ref[...]
    h_a = _leaky(_dot(embb, wa1cT_ref[...]) + base_a)
    pa = _dot(h_a.astype(jnp.bfloat16), wa2T_ref[...]) + ba2_ref[...]
    pa_out[...] = pa

    def mbody(seg, _):
        mask = bid == seg
        pam = jnp.where(mask, pa, NEG_INF)
        mblk = jnp.max(pam, axis=0, keepdims=True)
        mold = m_s[pl.ds(seg, 1), :]
        mnew = jnp.maximum(mold, mblk)
        scale = jnp.where(mold == NEG_INF, 0.0, jnp.exp(mold - mnew))
        e = jnp.where(mask, jnp.exp(pa - mnew), 0.0)
        s_s[pl.ds(seg, 1), :] = (s_s[pl.ds(seg, 1), :] * scale
                                 + jnp.sum(e, axis=0, keepdims=True))
        v_s[pl.ds(seg, 1), :] = (v_s[pl.ds(seg, 1), :] * scale
                                 + jnp.sum(e * raw, axis=0, keepdims=True))
        m_s[pl.ds(seg, 1), :] = mnew
        return 0

    lax.fori_loop(lo, hi + 1, mbody, 0)

    @pl.when(i == nb - 1)
    def _():
        r = 1.0 / (s_s[...] + 1e-6)
        vec = v_s[...] * r
        vec_out[...] = vec
        m_out[...] = m_s[...]
        r_out[...] = r
        h = _leaky(_dot(vec.astype(jnp.bfloat16), wo1T_ref[...]) + bo1_ref[...])
        aff_out[...] = _dot(h.astype(jnp.bfloat16), wo2T_ref[...]) + bo2_ref[...]


def _alpha_kernel(lo_ref, hi_ref, pa_ref, bid_ref, m_ref, r_ref, alpha_ref):
    i = pl.program_id(0)
    pa = pa_ref[...]
    bid = bid_ref[...]
    lo = lo_ref[i]
    hi = hi_ref[i]
    zero = jnp.zeros((CH, 128), jnp.float32)

    def body(seg, carry):
        mrow, rrow = carry
        mask = bid == seg
        mrow = jnp.where(mask, m_ref[pl.ds(seg, 1), :], mrow)
        rrow = jnp.where(mask, r_ref[pl.ds(seg, 1), :], rrow)
        return mrow, rrow

    mrow, rrow = lax.fori_loop(lo, hi + 1, body, (zero, zero))
    alpha = jnp.exp(pa - mrow) * rrow
    alpha_ref[...] = alpha[:, 0:1]


def _row_spec(ch, w):
    return pl.BlockSpec((ch, w), lambda i, *_: (i, 0))


def _full_spec(shape):
    return pl.BlockSpec(shape, lambda i, *_: tuple(0 for _ in shape))


def kernel(comp_feature, prot_feature, batch_comp, batch_prot,
           Wc, bc, Ws, bs, Wp, bp, Wr1, br1, Wr2, br2,
           Wa1, ba1, Wa2, ba2, Wo1, bo1, Wo2, bo2):
    n_c, h = comp_feature.shape
    n_p, _ = prot_feature.shape
    nb_c = n_c // CH
    nb_p = n_p // CH
    f32 = jnp.float32
    bf16 = jnp.bfloat16

    bid_c = batch_comp.astype(jnp.int32).reshape(n_c, 1)
    bid_p = batch_prot.astype(jnp.int32).reshape(n_p, 1)
    lo_c = bid_c[0::CH, 0]
    hi_c = bid_c[CH - 1::CH, 0]
    lo_p = bid_p[0::CH, 0]
    hi_p = bid_p[CH - 1::CH, 0]

    def row2(v):
        return v.reshape(1, -1).astype(f32)

    wpT = Wp.T.astype(bf16)
    wsT = Ws.T.astype(bf16)
    wcT = Wc.T.astype(bf16)
    wr1cT = Wr1[:, :h].T.astype(bf16)
    wr1sT = Wr1[:, h:2 * h].T.astype(bf16)
    wr1pT = Wr1[:, 2 * h:].T.astype(bf16)
    wr2T = Wr2.T.astype(bf16)
    wa1cT = Wa1[:, :h].T.astype(bf16)
    wa1sT = Wa1[:, h:2 * h].T.astype(bf16)
    wa1pT = Wa1[:, 2 * h:].T.astype(bf16)
    wo1T = Wo1.T.astype(bf16)
    wa2T = Wa2.T.astype(bf16)
    wo2T = Wo2.T.astype(bf16)
    ba2r = ba2.reshape(1, 1).astype(f32)
    bo2r = bo2.reshape(1, 1).astype(f32)

    prot_seg = pl.pallas_call(
        _prot_kernel,
        grid_spec=pltpu.PrefetchScalarGridSpec(
            num_scalar_prefetch=2,
            grid=(nb_p,),
            in_specs=[_row_spec(CH, h), _row_spec(CH, 1),
                      _full_spec((h, h)), _full_spec((1, h))],
            out_specs=_full_spec((B_SEG, h)),
        ),
        out_shape=jax.ShapeDtypeStruct((B_SEG, h), f32),
    )(lo_p, hi_p, prot_feature, bid_p, wpT, row2(bp))

    sup_seg = pl.pallas_call(
        _sup_kernel,
        grid_spec=pltpu.PrefetchScalarGridSpec(
            num_scalar_prefetch=2,
            grid=(nb_c,),
            in_specs=[_row_spec(CH, h), _row_spec(CH, 1),
                      _full_spec((h, h)), _full_spec((1, h))],
            out_specs=_full_spec((B_SEG, h)),
        ),
        out_shape=jax.ShapeDtypeStruct((B_SEG, h), f32),
    )(lo_c, hi_c, comp_feature, bid_c, wsT, row2(bs))

    pa, vector, affinity, m_arr, r_arr = pl.pallas_call(
        _main_kernel,
        grid_spec=pltpu.PrefetchScalarGridSpec(
            num_scalar_prefetch=2,
            grid=(nb_c,),
            in_specs=[_row_spec(CH, h), _row_spec(CH, 1),
                      _full_spec((h, h)), _full_spec((1, h)),
                      _full_spec((h, h)), _full_spec((h, h)), _full_spec((h, h)),
                      _full_spec((1, h)), _full_spec((h, h)), _full_spec((1, h)),
                      _full_spec((h, h)), _full_spec((h, h)), _full_spec((h, h)),
                      _full_spec((1, h)), _full_spec((h, 1)), _full_spec((1, 1)),
                      _full_spec((h, h)), _full_spec((1, h)), _full_spec((h, 1)),
                      _full_spec((1, 1)),
                      _full_spec((B_SEG, h)), _full_spec((B_SEG, h))],
            out_specs=[_row_spec(CH, 1),
                       _full_spec((B_SEG, h)),
                       _full_spec((B_SEG, 1)),
                       _full_spec((B_SEG, h)),
                       _full_spec((B_SEG, h))],
            scratch_shapes=[pltpu.VMEM((B_SEG, h), f32),
                            pltpu.VMEM((B_SEG, h), f32),
                            pltpu.VMEM((B_SEG, h), f32),
                            pltpu.VMEM((B_SEG, h), f32),
                            pltpu.VMEM((B_SEG, h), f32)],
        ),
        out_shape=[jax.ShapeDtypeStruct((n_c, 1), f32),
                   jax.ShapeDtypeStruct((B_SEG, h), f32),
                   jax.ShapeDtypeStruct((B_SEG, 1), f32),
                   jax.ShapeDtypeStruct((B_SEG, h), f32),
                   jax.ShapeDtypeStruct((B_SEG, h), f32)],
    )(lo_c, hi_c, comp_feature, bid_c,
      wcT, row2(bc),
      wr1cT, wr1sT, wr1pT, row2(br1), wr2T, row2(br2),
      wa1cT, wa1sT, wa1pT, row2(ba1), wa2T, ba2r,
      wo1T, row2(bo1), wo2T, bo2r,
      sup_seg, prot_seg)

    alpha = pl.pallas_call(
        _alpha_kernel,
        grid_spec=pltpu.PrefetchScalarGridSpec(
            num_scalar_prefetch=2,
            grid=(nb_c,),
            in_specs=[_row_spec(CH, 1), _row_spec(CH, 1),
                      _full_spec((B_SEG, h)), _full_spec((B_SEG, h))],
            out_specs=_row_spec(CH, 1),
        ),
        out_shape=jax.ShapeDtypeStruct((n_c, 1), f32),
    )(lo_c, hi_c, pa, bid_c, m_arr, r_arr)

    return vector, alpha, affinity
